# baseline (speedup 1.0000x reference)
import functools

import jax
import jax.numpy as jnp
from jax.experimental import pallas as pl
from jax.experimental.pallas import tpu as pltpu

_NBUF = 6
_BN = 256
_FLUSH = 4


def _gcn_body(x_ref, adj_hbm, w1_ref, b1_ref, w2_ref, b2_ref,
              feat_hbm, out_hbm,
              abuf, featbuf, outbuf, support_ref,
              in_sems, f_sems, o_sem, *, n_chunks):

    def adj_cp(k):
        return pltpu.make_async_copy(
            adj_hbm.at[pl.ds(k * _BN, _BN), :], abuf.at[k % _NBUF],
            in_sems.at[k % _NBUF])

    fl_rows = _FLUSH * _BN

    def feat_cp(q):
        return pltpu.make_async_copy(
            featbuf.at[pl.ds(q * fl_rows, fl_rows), :],
            feat_hbm.at[pl.ds(q * fl_rows, fl_rows), :],
            f_sems.at[q])

    out_cp = pltpu.make_async_copy(outbuf, out_hbm, o_sem)

    for k in range(min(_NBUF, n_chunks)):
        adj_cp(k).start()

    support_ref[...] = (
        jnp.dot(x_ref[...].astype(jnp.bfloat16),
                w1_ref[...].astype(jnp.bfloat16),
                preferred_element_type=jnp.float32)
        + b1_ref[...]
    ).astype(jnp.bfloat16)

    for k in range(n_chunks):
        adj_cp(k).wait()
        h = jnp.dot(abuf[k % _NBUF].astype(jnp.bfloat16), support_ref[...],
                    preferred_element_type=jnp.float32)
        if k + _NBUF < n_chunks:
            adj_cp(k + _NBUF).start()
        feat = jnp.maximum(h, 0.0)
        featbuf[pl.ds(k * _BN, _BN), :] = feat
        outbuf[pl.ds(k * _BN, _BN), :] = jax.nn.sigmoid(
            jnp.dot(feat.astype(jnp.bfloat16), w2_ref[...].astype(jnp.bfloat16),
                    preferred_element_type=jnp.float32)
            + b2_ref[...]
        )
        if (k + 1) % _FLUSH == 0:
            feat_cp((k + 1) // _FLUSH - 1).start()

    out_cp.start()
    for q in range(n_chunks // _FLUSH):
        feat_cp(q).wait()
    out_cp.wait()


@jax.jit
def _gcn_fused(x, adj, W1, b1, W2, b2):
    n, f = x.shape
    h_dim = W1.shape[1]
    c = W2.shape[1]
    n_chunks = n // _BN
    b1r = b1.reshape(1, h_dim)
    b2r = b2.reshape(1, c)
    feature, out = pl.pallas_call(
        functools.partial(_gcn_body, n_chunks=n_chunks),
        in_specs=[
            pl.BlockSpec(memory_space=pltpu.MemorySpace.VMEM),
            pl.BlockSpec(memory_space=pltpu.MemorySpace.HBM),
            pl.BlockSpec(memory_space=pltpu.MemorySpace.VMEM),
            pl.BlockSpec(memory_space=pltpu.MemorySpace.VMEM),
            pl.BlockSpec(memory_space=pltpu.MemorySpace.VMEM),
            pl.BlockSpec(memory_space=pltpu.MemorySpace.VMEM),
        ],
        out_specs=[
            pl.BlockSpec(memory_space=pltpu.MemorySpace.HBM),
            pl.BlockSpec(memory_space=pltpu.MemorySpace.HBM),
        ],
        out_shape=[
            jax.ShapeDtypeStruct((n, h_dim), jnp.float32),
            jax.ShapeDtypeStruct((n, c), jnp.float32),
        ],
        scratch_shapes=[
            pltpu.VMEM((_NBUF, _BN, n), jnp.float32),
            pltpu.VMEM((n, h_dim), jnp.float32),
            pltpu.VMEM((n, c), jnp.float32),
            pltpu.VMEM((n, h_dim), jnp.bfloat16),
            pltpu.SemaphoreType.DMA((_NBUF,)),
            pltpu.SemaphoreType.DMA((4,)),
            pltpu.SemaphoreType.DMA,
        ],
    )(x, adj, W1, b1r, W2, b2r)
    return feature, out


def kernel(x, adj, W1, b1, W2, b2):
    return _gcn_fused(x, adj, W1, b1, W2, b2)


# speedup vs baseline: 1.0328x; 1.0328x over previous
import functools

import jax
import jax.numpy as jnp
from jax.experimental import pallas as pl
from jax.experimental.pallas import tpu as pltpu


def _gcn_body(x_ref, adj_ref, w1_ref, b1_ref, w2_ref, b2_ref,
              feat_ref, out_ref, support_ref, hacc_ref, *, bk):
    i = pl.program_id(0)
    j = pl.program_id(1)

    @pl.when((i == 0) & (j == 0))
    def _compute_support():
        support_ref[...] = (
            jnp.dot(x_ref[...].astype(jnp.bfloat16),
                    w1_ref[...].astype(jnp.bfloat16),
                    preferred_element_type=jnp.float32)
            + b1_ref[...]
        ).astype(jnp.bfloat16)

    @pl.when(j == 0)
    def _first_half():
        hacc_ref[...] = jnp.dot(adj_ref[...].astype(jnp.bfloat16),
                                support_ref[:bk, :],
                                preferred_element_type=jnp.float32)

    @pl.when(j == 1)
    def _finalize():
        h = hacc_ref[...] + jnp.dot(adj_ref[...].astype(jnp.bfloat16),
                                    support_ref[bk:, :],
                                    preferred_element_type=jnp.float32)
        feat = jnp.maximum(h, 0.0)
        feat_ref[...] = feat
        out_ref[...] = jax.nn.sigmoid(
            jnp.dot(feat.astype(jnp.bfloat16), w2_ref[...].astype(jnp.bfloat16),
                    preferred_element_type=jnp.float32)
            + b2_ref[...]
        )


@functools.partial(jax.jit, static_argnames=("block_n",))
def _gcn_fused(x, adj, W1, b1, W2, b2, block_n=1024):
    n, f = x.shape
    h_dim = W1.shape[1]
    c = W2.shape[1]
    bk = n // 2
    b1r = b1.reshape(1, h_dim)
    b2r = b2.reshape(1, c)
    feature, out = pl.pallas_call(
        functools.partial(_gcn_body, bk=bk),
        grid=(n // block_n, 2),
        in_specs=[
            pl.BlockSpec((n, f), lambda i, j: (0, 0)),
            pl.BlockSpec((block_n, bk), lambda i, j: (i, j)),
            pl.BlockSpec((f, h_dim), lambda i, j: (0, 0)),
            pl.BlockSpec((1, h_dim), lambda i, j: (0, 0)),
            pl.BlockSpec((h_dim, c), lambda i, j: (0, 0)),
            pl.BlockSpec((1, c), lambda i, j: (0, 0)),
        ],
        out_specs=[
            pl.BlockSpec((block_n, h_dim), lambda i, j: (i, 0)),
            pl.BlockSpec((block_n, c), lambda i, j: (i, 0)),
        ],
        out_shape=[
            jax.ShapeDtypeStruct((n, h_dim), jnp.float32),
            jax.ShapeDtypeStruct((n, c), jnp.float32),
        ],
        scratch_shapes=[
            pltpu.VMEM((n, h_dim), jnp.bfloat16),
            pltpu.VMEM((block_n, h_dim), jnp.float32),
        ],
        compiler_params=pltpu.CompilerParams(
            dimension_semantics=("arbitrary", "arbitrary"),
        ),
    )(x, adj, W1, b1r, W2, b2r)
    return feature, out


def kernel(x, adj, W1, b1, W2, b2):
    return _gcn_fused(x, adj, W1, b1, W2, b2)
